# Initial kernel scaffold; baseline (speedup 1.0000x reference)
import jax
import jax.numpy as jnp
from jax.experimental import pallas as pl


def kernel(local_node_features, send_local_idx, edge_index, W_self, W_neigh, b):
    raise NotImplementedError("write your pallas kernel here")


# speedup vs baseline: 4.4319x; 4.4319x over previous
import jax
import jax.numpy as jnp
from jax import lax
from jax.experimental import pallas as pl
from jax.experimental.pallas import tpu as pltpu
from jax.experimental.pallas import tpu_sc as plsc

NUM_LOCAL = 10000
NUM_HALO = 2000
NUM_EDGES = 320000
D = 128

NC = 2
NS = 16
NW = NC * NS
L = 16

CHUNK = 128
NCHUNK = -(-NUM_EDGES // (NW * CHUNK))
PER_TILE = NCHUNK * CHUNK
E_PAD = PER_TILE * NW

LOC_PER_TILE = 632
NUM_LOCAL_PAD = LOC_PER_TILE * NS
HALO_SHIFT = NUM_LOCAL_PAD - NUM_LOCAL
HALO_PER_TILE = 128
NUM_HALO_PAD = HALO_PER_TILE * NS
XF_ROWS = NUM_LOCAL_PAD + NUM_HALO_PAD

ROWS_PER_TILE = 752
NUM_TOTAL_PAD = ROWS_PER_TILE * NS


def _staged_copy(src_ref, s0, dst_ref, d0, n, buf):
    done = 0
    while done < n:
        c = min(CHUNK, n - done)
        pltpu.sync_copy(src_ref.at[pl.ds(s0 + done, c)], buf.at[pl.ds(0, c)])
        pltpu.sync_copy(buf.at[pl.ds(0, c)], dst_ref.at[pl.ds(d0 + done, c)])
        done += c


def _sc_body(src_hbm, dst_hbm, slx_hbm, x_hbm, zero_hbm, agg_hbm, xf_hbm,
             idx_v, src_v, dst_v, rows_v, sem, acc_sh):
    cid = lax.axis_index("c")
    sid = lax.axis_index("s")
    wid = sid * NC + cid
    xbase = cid * XF_ROWS

    pltpu.sync_copy(zero_hbm, rows_v)
    done = 0
    while done < ROWS_PER_TILE:
        c = min(CHUNK, ROWS_PER_TILE - done)
        pltpu.sync_copy(rows_v.at[pl.ds(0, c)],
                        acc_sh.at[pl.ds(sid * ROWS_PER_TILE + done, c)])
        done += c

    r0 = sid * LOC_PER_TILE
    _staged_copy(x_hbm, r0, xf_hbm, xbase + r0, LOC_PER_TILE, rows_v)
    h0 = sid * HALO_PER_TILE
    pltpu.sync_copy(slx_hbm.at[pl.ds(h0, HALO_PER_TILE)], idx_v)
    pltpu.async_copy(x_hbm.at[idx_v], rows_v, sem).wait()
    pltpu.sync_copy(rows_v, xf_hbm.at[pl.ds(xbase + NUM_LOCAL_PAD + h0,
                                            HALO_PER_TILE)])
    plsc.subcore_barrier()

    base = wid * PER_TILE

    def chunk_body(c, carry):
        off = base + c * CHUNK
        pltpu.sync_copy(src_hbm.at[pl.ds(off, CHUNK)], src_v)
        pltpu.sync_copy(dst_hbm.at[pl.ds(off, CHUNK)], dst_v)
        for i in range(CHUNK // L):
            s = src_v[pl.ds(i * L, L)]
            s = jnp.where(s >= NUM_LOCAL, s + HALO_SHIFT, s) + xbase
            src_v[pl.ds(i * L, L)] = s
        pltpu.async_copy(xf_hbm.at[src_v], rows_v, sem).wait()
        pltpu.sync_copy(rows_v, acc_sh.at[dst_v], add=True)
        return carry

    lax.fori_loop(0, NCHUNK, chunk_body, 0)
    plsc.subcore_barrier()

    a0 = sid * ROWS_PER_TILE
    done = 0
    while done < ROWS_PER_TILE:
        c = min(CHUNK, ROWS_PER_TILE - done)
        pltpu.sync_copy(acc_sh.at[pl.ds(a0 + done, c)], rows_v.at[pl.ds(0, c)])
        pltpu.sync_copy(rows_v.at[pl.ds(0, c)],
                        agg_hbm.at[cid, pl.ds(a0 + done, c)])
        done += c


def _sc_message_pass(src_p, dst_p, slx_p, x_pad, zero_init):
    mesh = plsc.VectorSubcoreMesh(core_axis_name="c", subcore_axis_name="s")
    return pl.kernel(
        _sc_body,
        out_type=(
            jax.ShapeDtypeStruct((NC, NUM_TOTAL_PAD, D), jnp.float32),
            jax.ShapeDtypeStruct((NC * XF_ROWS, D), jnp.float32),
        ),
        mesh=mesh,
        scratch_types=[
            pltpu.VMEM((HALO_PER_TILE,), jnp.int32),
            pltpu.VMEM((CHUNK,), jnp.int32),
            pltpu.VMEM((CHUNK,), jnp.int32),
            pltpu.VMEM((CHUNK, D), jnp.float32),
            pltpu.SemaphoreType.DMA,
            pltpu.VMEM_SHARED((NUM_TOTAL_PAD, D), jnp.float32),
        ],
    )(src_p, dst_p, slx_p, x_pad, zero_init)


def _tc_body(x_ref, a0_ref, a1_ref, ws_ref, wn_ref, b_ref, o_ref):
    agg = a0_ref[0] + a1_ref[0]
    o_ref[...] = (
        jnp.dot(x_ref[...], ws_ref[...], preferred_element_type=jnp.float32)
        + jnp.dot(agg, wn_ref[...], preferred_element_type=jnp.float32)
        + b_ref[...]
    )


def _tc_combine(x, aggp, W_self, W_neigh, b2):
    blk = 1000
    grid = (NUM_LOCAL // blk,)
    return pl.pallas_call(
        _tc_body,
        grid=grid,
        in_specs=[
            pl.BlockSpec((blk, D), lambda i: (i, 0)),
            pl.BlockSpec((1, blk, D), lambda i: (0, i, 0)),
            pl.BlockSpec((1, blk, D), lambda i: (1, i, 0)),
            pl.BlockSpec((D, D), lambda i: (0, 0)),
            pl.BlockSpec((D, D), lambda i: (0, 0)),
            pl.BlockSpec((1, D), lambda i: (0, 0)),
        ],
        out_specs=pl.BlockSpec((blk, D), lambda i: (i, 0)),
        out_shape=jax.ShapeDtypeStruct((NUM_LOCAL, D), jnp.float32),
    )(x, aggp, aggp, W_self, W_neigh, b2)


@jax.jit
def kernel(local_node_features, send_local_idx, edge_index, W_self, W_neigh, b):
    src = edge_index[:, 0].astype(jnp.int32)
    dst = edge_index[:, 1].astype(jnp.int32)
    pad = E_PAD - NUM_EDGES
    src_p = jnp.concatenate([src, jnp.zeros((pad,), jnp.int32)])
    dst_p = jnp.concatenate([dst, jnp.full((pad,), NUM_LOCAL, jnp.int32)])
    slx_p = jnp.concatenate([
        send_local_idx.astype(jnp.int32),
        jnp.zeros((NUM_HALO_PAD - NUM_HALO,), jnp.int32),
    ])
    x_pad = jnp.concatenate([
        local_node_features,
        jnp.zeros((NUM_LOCAL_PAD - NUM_LOCAL, D), jnp.float32),
    ])
    zero_init = jnp.zeros((CHUNK, D), jnp.float32)
    aggp, _ = _sc_message_pass(src_p, dst_p, slx_p, x_pad, zero_init)
    return _tc_combine(local_node_features, aggp, W_self, W_neigh,
                       b.reshape(1, D))


# speedup vs baseline: 8.2719x; 1.8664x over previous
import jax
import jax.numpy as jnp
from jax import lax
from jax.experimental import pallas as pl
from jax.experimental.pallas import tpu as pltpu
from jax.experimental.pallas import tpu_sc as plsc

NUM_LOCAL = 10000
NUM_HALO = 2000
NUM_EDGES = 320000
D = 128

NC = 2
NS = 16
NW = NC * NS
L = 16

PER_TILE = NUM_EDGES // NW
CHUNK = 128
NPAIR = PER_TILE // (2 * CHUNK)
TAIL = PER_TILE - NPAIR * 2 * CHUNK

HALO_PER_TILE = 128
NUM_HALO_PAD = HALO_PER_TILE * NS
XF_ROWS = NUM_LOCAL + NUM_HALO_PAD
LOC_PER_TILE = 624
LOC_REM = NUM_LOCAL - LOC_PER_TILE * NS

ROWS_PER_TILE = 752
NUM_TOTAL_PAD = ROWS_PER_TILE * NS


def _sc_body(src_hbm, dst_hbm, slx_hbm, x_hbm, zero_hbm, agg_hbm, xf_hbm,
             idx_v, src_a, dst_a, src_b, dst_b, src_t, dst_t,
             rows_a, rows_b, sem_ga, sem_gb, sem_sa, sem_sb, acc_sh):
    cid = lax.axis_index("c")
    sid = lax.axis_index("s")
    wid = sid * NC + cid
    xbase = cid * XF_ROWS

    pltpu.sync_copy(zero_hbm, rows_a)
    done = 0
    while done < ROWS_PER_TILE:
        c = min(CHUNK, ROWS_PER_TILE - done)
        pltpu.sync_copy(rows_a.at[pl.ds(0, c)],
                        acc_sh.at[pl.ds(sid * ROWS_PER_TILE + done, c)])
        done += c

    r0 = sid * LOC_PER_TILE
    done = 0
    while done < LOC_PER_TILE:
        c = min(CHUNK, LOC_PER_TILE - done)
        pltpu.sync_copy(x_hbm.at[pl.ds(r0 + done, c)], rows_a.at[pl.ds(0, c)])
        pltpu.sync_copy(rows_a.at[pl.ds(0, c)],
                        xf_hbm.at[pl.ds(xbase + r0 + done, c)])
        done += c

    @pl.when(sid == NS - 1)
    def _():
        base = LOC_PER_TILE * NS
        pltpu.sync_copy(x_hbm.at[pl.ds(base, LOC_REM)],
                        rows_b.at[pl.ds(0, LOC_REM)])
        pltpu.sync_copy(rows_b.at[pl.ds(0, LOC_REM)],
                        xf_hbm.at[pl.ds(xbase + base, LOC_REM)])

    h0 = sid * HALO_PER_TILE
    pltpu.sync_copy(slx_hbm.at[pl.ds(h0, HALO_PER_TILE)], idx_v)
    pltpu.async_copy(x_hbm.at[idx_v], rows_a.at[pl.ds(0, HALO_PER_TILE)],
                     sem_ga).wait()
    pltpu.sync_copy(rows_a.at[pl.ds(0, HALO_PER_TILE)],
                    xf_hbm.at[pl.ds(xbase + NUM_LOCAL + h0, HALO_PER_TILE)])
    plsc.subcore_barrier()

    base = wid * PER_TILE

    def remap(sv):
        for i in range(CHUNK // L):
            sv[pl.ds(i * L, L)] = sv[pl.ds(i * L, L)] + xbase

    def pair_body(k, carry):
        offa = base + k * (2 * CHUNK)
        offb = offa + CHUNK
        pltpu.sync_copy(src_hbm.at[pl.ds(offa, CHUNK)], src_a)
        remap(src_a)

        pltpu.sync_copy(dst_hbm.at[pl.ds(offa, CHUNK)], dst_a)
        ga = pltpu.async_copy(xf_hbm.at[src_a], rows_a, sem_ga)
        pltpu.sync_copy(src_hbm.at[pl.ds(offb, CHUNK)], src_b)
        remap(src_b)

        pltpu.sync_copy(dst_hbm.at[pl.ds(offb, CHUNK)], dst_b)
        gb = pltpu.async_copy(xf_hbm.at[src_b], rows_b, sem_gb)
        ga.wait()
        pltpu.sync_copy(rows_a, acc_sh.at[dst_a], add=True)
        gb.wait()
        pltpu.sync_copy(rows_b, acc_sh.at[dst_b], add=True)
        return carry

    lax.fori_loop(0, NPAIR, pair_body, 0)

    offt = base + NPAIR * 2 * CHUNK
    pltpu.sync_copy(src_hbm.at[pl.ds(offt, TAIL)], src_t)
    pltpu.sync_copy(dst_hbm.at[pl.ds(offt, TAIL)], dst_t)
    src_t[...] = src_t[...] + xbase
    pltpu.async_copy(xf_hbm.at[src_t], rows_a.at[pl.ds(0, TAIL)],
                     sem_ga).wait()
    pltpu.sync_copy(rows_a.at[pl.ds(0, TAIL)], acc_sh.at[dst_t], add=True)
    plsc.subcore_barrier()

    a0 = sid * ROWS_PER_TILE
    done = 0
    while done < ROWS_PER_TILE:
        c = min(CHUNK, ROWS_PER_TILE - done)
        pltpu.sync_copy(acc_sh.at[pl.ds(a0 + done, c)], rows_a.at[pl.ds(0, c)])
        pltpu.sync_copy(rows_a.at[pl.ds(0, c)],
                        agg_hbm.at[cid, pl.ds(a0 + done, c)])
        done += c


def _sc_message_pass(src, dst, slx_p, x, zero_init):
    mesh = plsc.VectorSubcoreMesh(core_axis_name="c", subcore_axis_name="s")
    return pl.kernel(
        _sc_body,
        out_type=(
            jax.ShapeDtypeStruct((NC, NUM_TOTAL_PAD, D), jnp.float32),
            jax.ShapeDtypeStruct((NC * XF_ROWS, D), jnp.float32),
        ),
        mesh=mesh,
        scratch_types=[
            pltpu.VMEM((HALO_PER_TILE,), jnp.int32),
            pltpu.VMEM((CHUNK,), jnp.int32),
            pltpu.VMEM((CHUNK,), jnp.int32),
            pltpu.VMEM((CHUNK,), jnp.int32),
            pltpu.VMEM((CHUNK,), jnp.int32),
            pltpu.VMEM((TAIL,), jnp.int32),
            pltpu.VMEM((TAIL,), jnp.int32),
            pltpu.VMEM((CHUNK, D), jnp.float32),
            pltpu.VMEM((CHUNK, D), jnp.float32),
            pltpu.SemaphoreType.DMA,
            pltpu.SemaphoreType.DMA,
            pltpu.SemaphoreType.DMA,
            pltpu.SemaphoreType.DMA,
            pltpu.VMEM_SHARED((NUM_TOTAL_PAD, D), jnp.float32),
        ],
    )(src, dst, slx_p, x, zero_init)


def _tc_body(x_ref, a0_ref, a1_ref, ws_ref, wn_ref, b_ref, o_ref):
    agg = a0_ref[0] + a1_ref[0]
    o_ref[...] = (
        jnp.dot(x_ref[...], ws_ref[...], preferred_element_type=jnp.float32)
        + jnp.dot(agg, wn_ref[...], preferred_element_type=jnp.float32)
        + b_ref[...]
    )


def _tc_combine(x, aggp, W_self, W_neigh, b2):
    blk = 1000
    grid = (NUM_LOCAL // blk,)
    return pl.pallas_call(
        _tc_body,
        grid=grid,
        in_specs=[
            pl.BlockSpec((blk, D), lambda i: (i, 0)),
            pl.BlockSpec((1, blk, D), lambda i: (0, i, 0)),
            pl.BlockSpec((1, blk, D), lambda i: (1, i, 0)),
            pl.BlockSpec((D, D), lambda i: (0, 0)),
            pl.BlockSpec((D, D), lambda i: (0, 0)),
            pl.BlockSpec((1, D), lambda i: (0, 0)),
        ],
        out_specs=pl.BlockSpec((blk, D), lambda i: (i, 0)),
        out_shape=jax.ShapeDtypeStruct((NUM_LOCAL, D), jnp.float32),
    )(x, aggp, aggp, W_self, W_neigh, b2)


@jax.jit
def kernel(local_node_features, send_local_idx, edge_index, W_self, W_neigh, b):
    src = edge_index[:, 0].astype(jnp.int32)
    dst = edge_index[:, 1].astype(jnp.int32)
    slx_p = jnp.concatenate([
        send_local_idx.astype(jnp.int32),
        jnp.zeros((NUM_HALO_PAD - NUM_HALO,), jnp.int32),
    ])
    zero_init = jnp.zeros((CHUNK, D), jnp.float32)
    aggp, _ = _sc_message_pass(src, dst, slx_p, local_node_features, zero_init)
    return _tc_combine(local_node_features, aggp, W_self, W_neigh,
                       b.reshape(1, D))


# speedup vs baseline: 9.7844x; 1.1828x over previous
import jax
import jax.numpy as jnp
from jax import lax
from jax.experimental import pallas as pl
from jax.experimental.pallas import tpu as pltpu
from jax.experimental.pallas import tpu_sc as plsc

NUM_LOCAL = 10000
NUM_HALO = 2000
NUM_EDGES = 320000
D = 128

NC = 2
NS = 16
NW = NC * NS
L = 16

PER_TILE = NUM_EDGES // NW
CHUNK = 128
NBUF = 3
NTRIP = PER_TILE // (NBUF * CHUNK)
TAIL = PER_TILE - NTRIP * NBUF * CHUNK

HALO_PER_TILE = 128
NUM_HALO_PAD = HALO_PER_TILE * NS
XF_ROWS = NUM_LOCAL + NUM_HALO_PAD
LOC_PER_TILE = 624
LOC_REM = NUM_LOCAL - LOC_PER_TILE * NS

ROWS_PER_TILE = 632
ACC_ROWS = ROWS_PER_TILE * NS
SCRAP_ROW = NUM_LOCAL


def _sc_body(src_hbm, dst_hbm, slx_hbm, x_hbm, zero_hbm, agg_hbm, xf_hbm,
             srcs, dsts, src_t, dst_t, rows, gsems, ssems, acc_sh):
    cid = lax.axis_index("c")
    sid = lax.axis_index("s")
    wid = sid * NC + cid
    xbase = cid * XF_ROWS

    pltpu.sync_copy(zero_hbm, rows[0])
    done = 0
    while done < ROWS_PER_TILE:
        c = min(CHUNK, ROWS_PER_TILE - done)
        pltpu.sync_copy(rows[0].at[pl.ds(0, c)],
                        acc_sh.at[pl.ds(sid * ROWS_PER_TILE + done, c)])
        done += c

    r0 = sid * LOC_PER_TILE
    done = 0
    while done < LOC_PER_TILE:
        c = min(CHUNK, LOC_PER_TILE - done)
        pltpu.sync_copy(x_hbm.at[pl.ds(r0 + done, c)], rows[0].at[pl.ds(0, c)])
        pltpu.sync_copy(rows[0].at[pl.ds(0, c)],
                        xf_hbm.at[pl.ds(xbase + r0 + done, c)])
        done += c

    @pl.when(sid == NS - 1)
    def _():
        lbase = LOC_PER_TILE * NS
        pltpu.sync_copy(x_hbm.at[pl.ds(lbase, LOC_REM)],
                        rows[1].at[pl.ds(0, LOC_REM)])
        pltpu.sync_copy(rows[1].at[pl.ds(0, LOC_REM)],
                        xf_hbm.at[pl.ds(xbase + lbase, LOC_REM)])

    h0 = sid * HALO_PER_TILE
    pltpu.sync_copy(slx_hbm.at[pl.ds(h0, HALO_PER_TILE)], srcs[0])
    pltpu.async_copy(x_hbm.at[srcs[0]], rows[0], gsems[0]).wait()
    pltpu.sync_copy(rows[0],
                    xf_hbm.at[pl.ds(xbase + NUM_LOCAL + h0, HALO_PER_TILE)])
    plsc.subcore_barrier()

    base = wid * PER_TILE

    def prep(sv, dv):
        for i in range(CHUNK // L):
            sv[pl.ds(i * L, L)] = sv[pl.ds(i * L, L)] + xbase
            dv[pl.ds(i * L, L)] = jnp.minimum(dv[pl.ds(i * L, L)], SCRAP_ROW)

    def trip_body(k, carry):
        descs = []
        for j in range(NBUF):
            off = base + (k * NBUF + j) * CHUNK
            pltpu.sync_copy(src_hbm.at[pl.ds(off, CHUNK)], srcs[j])

            @pl.when(k > 0)
            def _(j=j):
                pltpu.make_async_copy(rows[j], acc_sh.at[dsts[j]],
                                      ssems[j]).wait()

            pltpu.sync_copy(dst_hbm.at[pl.ds(off, CHUNK)], dsts[j])
            prep(srcs[j], dsts[j])
            descs.append(pltpu.async_copy(xf_hbm.at[srcs[j]], rows[j],
                                          gsems[j]))
        for j in range(NBUF):
            descs[j].wait()
            pltpu.async_copy(rows[j], acc_sh.at[dsts[j]], ssems[j], add=True)
        return carry

    lax.fori_loop(0, NTRIP, trip_body, 0)
    for j in range(NBUF):
        pltpu.make_async_copy(rows[j], acc_sh.at[dsts[j]], ssems[j]).wait()

    offt = base + NTRIP * NBUF * CHUNK
    pltpu.sync_copy(src_hbm.at[pl.ds(offt, TAIL)], src_t)
    pltpu.sync_copy(dst_hbm.at[pl.ds(offt, TAIL)], dst_t)
    src_t[...] = src_t[...] + xbase
    dst_t[...] = jnp.minimum(dst_t[...], SCRAP_ROW)
    pltpu.async_copy(xf_hbm.at[src_t], rows[0].at[pl.ds(0, TAIL)],
                     gsems[0]).wait()
    pltpu.sync_copy(rows[0].at[pl.ds(0, TAIL)], acc_sh.at[dst_t], add=True)
    plsc.subcore_barrier()

    a0 = sid * ROWS_PER_TILE
    done = 0
    while done < ROWS_PER_TILE:
        c = min(CHUNK, ROWS_PER_TILE - done)
        pltpu.sync_copy(acc_sh.at[pl.ds(a0 + done, c)], rows[0].at[pl.ds(0, c)])
        pltpu.sync_copy(rows[0].at[pl.ds(0, c)],
                        agg_hbm.at[cid, pl.ds(a0 + done, c)])
        done += c


def _sc_message_pass(src, dst, slx_p, x, zero_init):
    mesh = plsc.VectorSubcoreMesh(core_axis_name="c", subcore_axis_name="s")
    return pl.kernel(
        _sc_body,
        out_type=(
            jax.ShapeDtypeStruct((NC, ACC_ROWS, D), jnp.float32),
            jax.ShapeDtypeStruct((NC * XF_ROWS, D), jnp.float32),
        ),
        mesh=mesh,
        scratch_types=[
            [pltpu.VMEM((CHUNK,), jnp.int32) for _ in range(NBUF)],
            [pltpu.VMEM((CHUNK,), jnp.int32) for _ in range(NBUF)],
            pltpu.VMEM((TAIL,), jnp.int32),
            pltpu.VMEM((TAIL,), jnp.int32),
            [pltpu.VMEM((CHUNK, D), jnp.float32) for _ in range(NBUF)],
            [pltpu.SemaphoreType.DMA for _ in range(NBUF)],
            [pltpu.SemaphoreType.DMA for _ in range(NBUF)],
            pltpu.VMEM_SHARED((ACC_ROWS, D), jnp.float32),
        ],
    )(src, dst, slx_p, x, zero_init)


def _tc_body(x_ref, a0_ref, a1_ref, ws_ref, wn_ref, b_ref, o_ref):
    agg = a0_ref[0] + a1_ref[0]
    o_ref[...] = (
        jnp.dot(x_ref[...], ws_ref[...], preferred_element_type=jnp.float32)
        + jnp.dot(agg, wn_ref[...], preferred_element_type=jnp.float32)
        + b_ref[...]
    )


def _tc_combine(x, aggp, W_self, W_neigh, b2):
    blk = 1000
    grid = (NUM_LOCAL // blk,)
    return pl.pallas_call(
        _tc_body,
        grid=grid,
        in_specs=[
            pl.BlockSpec((blk, D), lambda i: (i, 0)),
            pl.BlockSpec((1, blk, D), lambda i: (0, i, 0)),
            pl.BlockSpec((1, blk, D), lambda i: (1, i, 0)),
            pl.BlockSpec((D, D), lambda i: (0, 0)),
            pl.BlockSpec((D, D), lambda i: (0, 0)),
            pl.BlockSpec((1, D), lambda i: (0, 0)),
        ],
        out_specs=pl.BlockSpec((blk, D), lambda i: (i, 0)),
        out_shape=jax.ShapeDtypeStruct((NUM_LOCAL, D), jnp.float32),
    )(x, aggp, aggp, W_self, W_neigh, b2)


@jax.jit
def kernel(local_node_features, send_local_idx, edge_index, W_self, W_neigh, b):
    src = edge_index[:, 0].astype(jnp.int32)
    dst = edge_index[:, 1].astype(jnp.int32)
    slx_p = jnp.concatenate([
        send_local_idx.astype(jnp.int32),
        jnp.zeros((NUM_HALO_PAD - NUM_HALO,), jnp.int32),
    ])
    zero_init = jnp.zeros((CHUNK, D), jnp.float32)
    aggp, _ = _sc_message_pass(src, dst, slx_p, local_node_features, zero_init)
    return _tc_combine(local_node_features, aggp, W_self, W_neigh,
                       b.reshape(1, D))
